# baseline (speedup 1.0000x reference)
import jax
import jax.numpy as jnp
from jax.experimental import pallas as pl


def _add_body(x_ref, pos_ref, o_ref):
    o_ref[...] = x_ref[...] + pos_ref[...]


def kernel(x, pos):
    B, T, D = x.shape
    TBLK = 1024
    BBLK = 2
    nt = T // TBLK
    nb = B // BBLK
    return pl.pallas_call(
        _add_body,
        grid=(nt, nb),
        in_specs=[
            pl.BlockSpec((BBLK, TBLK, D), lambda t, b: (b, t, 0)),
            pl.BlockSpec((TBLK, D), lambda t, b: (t, 0)),
        ],
        out_specs=pl.BlockSpec((BBLK, TBLK, D), lambda t, b: (b, t, 0)),
        out_shape=jax.ShapeDtypeStruct(x.shape, x.dtype),
    )(x, pos)
